# Initial kernel scaffold; baseline (speedup 1.0000x reference)
import jax
import jax.numpy as jnp
from jax.experimental import pallas as pl


def kernel(x, offset):
    raise NotImplementedError("write your pallas kernel here")


# speedup vs baseline: 2.6702x; 2.6702x over previous
import functools

import jax
import jax.numpy as jnp
from jax import lax
from jax.experimental import pallas as pl
from jax.experimental.pallas import tpu as pltpu
from jax.experimental.pallas import tpu_sc as plsc

H = 224
W = 224
HW = H * W
C = 96
KH = 3
KW = 3
NTAP = KH * KW

NC = 2
NS = 16
NW = NC * NS
LANES = 16

PX_PER_TILE = HW // NW
G = 112
WINDOWS = PX_PER_TILE // G


def _transpose_body(x_ref, o_ref):
    o_ref[...] = x_ref[...].T


def _make_table(x2):
    blk = 512
    return pl.pallas_call(
        _transpose_body,
        grid=(HW // blk,),
        in_specs=[pl.BlockSpec((C, blk), lambda i: (0, i))],
        out_specs=pl.BlockSpec((blk, C), lambda i: (i, 0)),
        out_shape=jax.ShapeDtypeStruct((HW, C), jnp.float32),
    )(x2)


def _sc_body(xT, off2, ymap, xmap, out,
             offy_v, offx_v, ym_v, xm_v,
             i0, i1, i2, i3, w0, w1, w2, w3,
             g0, g1, g2, g3, out_v, sem):
    idx_refs = (i0, i1, i2, i3)
    w_refs = (w0, w1, w2, w3)
    g_refs = (g0, g1, g2, g3)
    cid = lax.axis_index("c")
    sid = lax.axis_index("s")
    wid = sid * NC + cid
    tile_base = wid * PX_PER_TILE
    iota16 = lax.iota(jnp.int32, LANES)

    @pl.loop(0, KH)
    def _(ti):
        @pl.loop(0, KW)
        def _(tj):
            t = ti * KW + tj
            dyf = (ti - 1).astype(jnp.float32)
            dxf = (tj - 1).astype(jnp.float32)

            @pl.loop(0, WINDOWS)
            def _(wi):
                p0 = tile_base + wi * G
                pltpu.sync_copy(off2.at[2 * t, pl.ds(p0, G)], offy_v)
                pltpu.sync_copy(off2.at[2 * t + 1, pl.ds(p0, G)], offx_v)
                pltpu.sync_copy(ymap.at[pl.ds(p0, G)], ym_v)
                pltpu.sync_copy(xmap.at[pl.ds(p0, G)], xm_v)

                @pl.loop(0, G // LANES)
                def _(ci):
                    s = ci * LANES
                    sl = pl.ds(s, LANES)
                    y = ym_v[sl] + offy_v[sl] + dyf
                    x = xm_v[sl] + offx_v[sl] + dxf
                    yt = y.astype(jnp.int32)
                    ytf = yt.astype(jnp.float32)
                    y0f = jnp.where(ytf > y, ytf - 1.0, ytf)
                    y0 = y0f.astype(jnp.int32)
                    xt = x.astype(jnp.int32)
                    xtf = xt.astype(jnp.float32)
                    x0f = jnp.where(xtf > x, xtf - 1.0, xtf)
                    x0 = x0f.astype(jnp.int32)
                    fy1 = y - y0f
                    fy0 = 1.0 - fy1
                    fx1 = x - x0f
                    fx0 = 1.0 - fx1
                    wy = (fy0, fy1)
                    wx = (fx0, fx1)
                    for k, (ky, kx) in enumerate(
                            ((0, 0), (0, 1), (1, 0), (1, 1))):
                        yc = y0 + ky
                        xc = x0 + kx
                        inb = ((yc >= 0) & (yc < H) & (xc >= 0) & (xc < W))
                        idx = jnp.where(inb, yc * W + xc, 0)
                        wgt = jnp.where(inb, wy[ky] * wx[kx], 0.0)
                        idx_refs[k][sl] = idx
                        w_refs[k][sl] = wgt

                cps = [pltpu.async_copy(xT.at[idx_refs[k]], g_refs[k], sem)
                       for k in range(4)]
                for cp in cps:
                    cp.wait()

                @pl.loop(0, G // LANES)
                def _(ci):
                    pb = ci * LANES
                    pbs = pl.ds(pb, LANES)
                    pidx = pb + iota16
                    wv0 = w0[pbs]
                    wv1 = w1[pbs]
                    wv2 = w2[pbs]
                    wv3 = w3[pbs]

                    @pl.loop(0, C, unroll=4)
                    def _(c):
                        cidx = lax.broadcast(c, (LANES,))
                        v0 = plsc.load_gather(g0, [pidx, cidx])
                        v1 = plsc.load_gather(g1, [pidx, cidx])
                        v2 = plsc.load_gather(g2, [pidx, cidx])
                        v3 = plsc.load_gather(g3, [pidx, cidx])
                        acc = v0 * wv0 + v1 * wv1 + v2 * wv2 + v3 * wv3
                        out_v[c, pbs] = acc

                pltpu.sync_copy(
                    out_v, out.at[pl.ds(t * C, C), pl.ds(p0, G)])


@jax.jit
def _deform_columns(xT, off2, ymap, xmap):
    mesh = plsc.VectorSubcoreMesh(core_axis_name="c", subcore_axis_name="s")
    f = pl.kernel(
        _sc_body,
        out_type=jax.ShapeDtypeStruct((NTAP * C, HW), jnp.float32),
        mesh=mesh,
        compiler_params=pltpu.CompilerParams(
            use_tc_tiling_on_sc=False, needs_layout_passes=False),
        scratch_types=[
            pltpu.VMEM((G,), jnp.float32),
            pltpu.VMEM((G,), jnp.float32),
            pltpu.VMEM((G,), jnp.float32),
            pltpu.VMEM((G,), jnp.float32),
            pltpu.VMEM((G,), jnp.int32),
            pltpu.VMEM((G,), jnp.int32),
            pltpu.VMEM((G,), jnp.int32),
            pltpu.VMEM((G,), jnp.int32),
            pltpu.VMEM((G,), jnp.float32),
            pltpu.VMEM((G,), jnp.float32),
            pltpu.VMEM((G,), jnp.float32),
            pltpu.VMEM((G,), jnp.float32),
            pltpu.VMEM((G, C), jnp.float32),
            pltpu.VMEM((G, C), jnp.float32),
            pltpu.VMEM((G, C), jnp.float32),
            pltpu.VMEM((G, C), jnp.float32),
            pltpu.VMEM((C, G), jnp.float32),
            pltpu.SemaphoreType.DMA,
        ],
    )
    return f(xT, off2, ymap, xmap)


def kernel(x, offset):
    x2 = x.reshape(C, HW)
    off2 = offset.reshape(2 * NTAP, HW)
    xT = _make_table(x2)
    ymap = jnp.broadcast_to(
        jnp.arange(H, dtype=jnp.float32)[:, None], (H, W)).reshape(HW)
    xmap = jnp.broadcast_to(
        jnp.arange(W, dtype=jnp.float32)[None, :], (H, W)).reshape(HW)
    cols = _deform_columns(xT, off2, ymap, xmap)
    return cols.reshape(1, NTAP * C, H, W)
